# Initial kernel scaffold; baseline (speedup 1.0000x reference)
import jax
import jax.numpy as jnp
from jax.experimental import pallas as pl


def kernel(pos_state, pos_action, h, x, u, a2s_edge_index, a2s_dis, s2s_edge_index, s2s_dis, u2h_W1, u2h_b1, u2h_W2, u2h_b2, u2h_W3, u2h_b3, x2h_W1, x2h_b1, x2h_W2, x2h_b2, x2h_W3, x2h_b3, upd_W1, upd_b1, upd_W2, upd_b2, upd_W3, upd_b3):
    raise NotImplementedError("write your pallas kernel here")


# speedup vs baseline: 3.3076x; 3.3076x over previous
import functools

import jax
import jax.numpy as jnp
from jax import lax
from jax.experimental import pallas as pl
from jax.experimental.pallas import tpu as pltpu
from jax.experimental.pallas import tpu_sc as plsc

N_NODE = 100000
E_EDGE = 1600000
NW = 32
IDX_LANES = 128
ROWS_PER_W = 400
E_PAD = NW * ROWS_PER_W * IDX_LANES
CHUNK_ROWS = 8
NCHUNK = ROWS_PER_W // CHUNK_ROWS
CHUNK_E = CHUNK_ROWS * IDX_LANES
ACC_PER_TILE = 6272
ACC_ROWS = ACC_PER_TILE * 16
NP = 102400
NB = 4096
F32 = jnp.float32


def _mesh():
    return plsc.VectorSubcoreMesh(core_axis_name="c", subcore_axis_name="s")


def _sc_gather(src_table, dst_table, sidx2, didx2):
    @functools.partial(
        pl.kernel,
        out_type=(
            jax.ShapeDtypeStruct((E_PAD, 32), F32),
            jax.ShapeDtypeStruct((E_PAD, 32), F32),
        ),
        mesh=_mesh(),
        scratch_types=[
            pltpu.VMEM((CHUNK_ROWS, IDX_LANES), jnp.int32),
            pltpu.VMEM((CHUNK_ROWS, IDX_LANES), jnp.int32),
            pltpu.VMEM((CHUNK_E, 32), F32),
            pltpu.VMEM((CHUNK_E, 32), F32),
            pltpu.SemaphoreType.DMA,
        ],
        compiler_params=pltpu.CompilerParams(use_tc_tiling_on_sc=False),
    )
    def k(srcT, dstT, sidx, didx, gs_out, gd_out, sidx_v, didx_v, gs_v, gd_v, sem):
        wid = lax.axis_index("s") * 2 + lax.axis_index("c")

        def body(ch, carry):
            rowbase = wid * ROWS_PER_W + ch * CHUNK_ROWS
            ebase = rowbase * IDX_LANES
            pltpu.sync_copy(sidx.at[pl.ds(rowbase, CHUNK_ROWS)], sidx_v)
            pltpu.sync_copy(didx.at[pl.ds(rowbase, CHUNK_ROWS)], didx_v)
            cps = []
            for j in range(CHUNK_ROWS):
                sl = pl.ds(j * IDX_LANES, IDX_LANES)
                cps.append(pltpu.async_copy(srcT.at[sidx_v.at[j]], gs_v.at[sl], sem))
                cps.append(pltpu.async_copy(dstT.at[didx_v.at[j]], gd_v.at[sl], sem))
            for c in cps:
                c.wait()
            pltpu.sync_copy(gs_v, gs_out.at[pl.ds(ebase, CHUNK_E)])
            pltpu.sync_copy(gd_v, gd_out.at[pl.ds(ebase, CHUNK_E)])
            return carry

        lax.fori_loop(0, NCHUNK, body, 0)

    return k(src_table, dst_table, sidx2, didx2)


def _sc_scatter(msg, didx2, zeros16, zeros1, with_counts):
    @functools.partial(
        pl.kernel,
        out_type=(
            jax.ShapeDtypeStruct((2, ACC_ROWS, 16), F32),
            jax.ShapeDtypeStruct((2, ACC_ROWS), F32),
        ),
        mesh=_mesh(),
        scratch_types=[
            pltpu.VMEM((CHUNK_ROWS, IDX_LANES), jnp.int32),
            pltpu.VMEM((CHUNK_E, 16), F32),
            pltpu.VMEM((IDX_LANES,), F32),
            pltpu.VMEM_SHARED((ACC_ROWS, 16), F32),
            pltpu.VMEM_SHARED((ACC_ROWS,), F32),
        ],
        compiler_params=pltpu.CompilerParams(use_tc_tiling_on_sc=False),
    )
    def k(msg_h, didx, z16, z1, sums_o, cnts_o, idx_v, msg_v, ones_v, accum, cnta):
        c = lax.axis_index("c")
        s = lax.axis_index("s")
        wid = s * 2 + c
        tb = s * ACC_PER_TILE
        pltpu.sync_copy(z16, accum.at[pl.ds(tb, ACC_PER_TILE)])
        pltpu.sync_copy(z1, cnta.at[pl.ds(tb, ACC_PER_TILE)])
        for i in range(IDX_LANES // 16):
            ones_v[pl.ds(i * 16, 16)] = jnp.full((16,), 1.0, F32)
        plsc.subcore_barrier()

        def body(ch, carry):
            rowbase = wid * ROWS_PER_W + ch * CHUNK_ROWS
            pltpu.sync_copy(msg_h.at[pl.ds(rowbase * IDX_LANES, CHUNK_E)], msg_v)
            pltpu.sync_copy(didx.at[pl.ds(rowbase, CHUNK_ROWS)], idx_v)
            for j in range(CHUNK_ROWS):
                row = idx_v.at[j]
                pltpu.sync_copy(msg_v.at[pl.ds(j * IDX_LANES, IDX_LANES)],
                                accum.at[row], add=True)
                if with_counts:
                    pltpu.sync_copy(ones_v, cnta.at[row], add=True)
            return carry

        lax.fori_loop(0, NCHUNK, body, 0)
        plsc.subcore_barrier()
        pltpu.sync_copy(accum.at[pl.ds(tb, ACC_PER_TILE)],
                        sums_o.at[c].at[pl.ds(tb, ACC_PER_TILE)])
        pltpu.sync_copy(cnta.at[pl.ds(tb, ACC_PER_TILE)],
                        cnts_o.at[c].at[pl.ds(tb, ACC_PER_TILE)])

    return k(msg, didx2, zeros16, zeros1)


def _full(shape):
    return pl.BlockSpec(shape, lambda i: (0,) * len(shape))


def _tc_project(featA, featS, ps, WA, bA, WS, bS, Wud, Wxd):
    grid = NP // NB

    def body(fa, fs, p, wa, ba, ws, bs_, wud, wxd, ta, ts, tdu, tdx):
        ta[...] = jnp.dot(fa[...], wa[...], preferred_element_type=F32) + ba[...]
        ts[...] = jnp.dot(fs[...], ws[...], preferred_element_type=F32) + bs_[...]
        tdu[...] = jnp.dot(p[...], wud[...], preferred_element_type=F32)
        tdx[...] = jnp.dot(p[...], wxd[...], preferred_element_type=F32)

    return pl.pallas_call(
        body,
        grid=(grid,),
        in_specs=[
            pl.BlockSpec((NB, 6), lambda i: (i, 0)),
            pl.BlockSpec((NB, 26), lambda i: (i, 0)),
            pl.BlockSpec((NB, 2), lambda i: (i, 0)),
            _full((6, 32)), _full((1, 32)),
            _full((26, 32)), _full((1, 32)),
            _full((2, 32)), _full((2, 32)),
        ],
        out_specs=[pl.BlockSpec((NB, 32), lambda i: (i, 0))] * 4,
        out_shape=[jax.ShapeDtypeStruct((NP, 32), F32)] * 4,
    )(featA, featS, ps, WA, bA, WS, bS, Wud, Wxd)


def _tc_edge_mlp(gs, gd, dis, w1d, W2, b2, W3, b3):
    grid = E_PAD // NB

    def body(a, b, d, wd, w2, b2_, w3, b3_, out):
        z = a[...] + b[...] + d[...] * wd[...]
        h1 = jnp.tanh(z)
        h2 = jnp.tanh(jnp.dot(h1, w2[...], preferred_element_type=F32) + b2_[...])
        out[...] = jnp.dot(h2, w3[...], preferred_element_type=F32) + b3_[...]

    return pl.pallas_call(
        body,
        grid=(grid,),
        in_specs=[
            pl.BlockSpec((NB, 32), lambda i: (i, 0)),
            pl.BlockSpec((NB, 32), lambda i: (i, 0)),
            pl.BlockSpec((NB, 1), lambda i: (i, 0)),
            _full((1, 32)),
            _full((32, 32)), _full((1, 32)),
            _full((32, 16)), _full((1, 16)),
        ],
        out_specs=pl.BlockSpec((NB, 16), lambda i: (i, 0)),
        out_shape=jax.ShapeDtypeStruct((E_PAD, 16), F32),
    )(gs, gd, dis, w1d, W2, b2, W3, b3)


def _tc_final(ps, h, x, su, nx, cn, Wp, Wh, Wsu, Wm, Wx, b1, W2, b2, W3, b3):
    grid = NP // NB

    def body(p, h_, x_, su_, nx_, cn_, wp, wh, wsu, wm, wx, b1_, w2, b2_, w3, b3_, out):
        sum_u = su_[0] + su_[1]
        num_x = nx_[0] + nx_[1]
        cnt = cn_[0] + cn_[1]
        mean_x = num_x / jnp.maximum(cnt, 1.0)[:, None]
        z = (jnp.dot(p[...], wp[...], preferred_element_type=F32)
             + jnp.dot(h_[...], wh[...], preferred_element_type=F32)
             + jnp.dot(sum_u, wsu[...], preferred_element_type=F32)
             + jnp.dot(mean_x, wm[...], preferred_element_type=F32)
             + jnp.dot(x_[...], wx[...], preferred_element_type=F32)
             + b1_[...])
        h1 = jnp.tanh(z)
        h2 = jnp.tanh(jnp.dot(h1, w2[...], preferred_element_type=F32) + b2_[...])
        out[...] = jnp.dot(h2, w3[...], preferred_element_type=F32) + b3_[...]

    return pl.pallas_call(
        body,
        grid=(grid,),
        in_specs=[
            pl.BlockSpec((NB, 2), lambda i: (i, 0)),
            pl.BlockSpec((NB, 16), lambda i: (i, 0)),
            pl.BlockSpec((NB, 8), lambda i: (i, 0)),
            pl.BlockSpec((2, NB, 16), lambda i: (0, i, 0)),
            pl.BlockSpec((2, NB, 16), lambda i: (0, i, 0)),
            pl.BlockSpec((2, NB), lambda i: (0, i)),
            _full((2, 32)), _full((16, 32)), _full((16, 32)),
            _full((16, 32)), _full((8, 32)), _full((1, 32)),
            _full((32, 32)), _full((1, 32)),
            _full((32, 16)), _full((1, 16)),
        ],
        out_specs=pl.BlockSpec((NB, 16), lambda i: (i, 0)),
        out_shape=jax.ShapeDtypeStruct((NP, 16), F32),
    )(ps, h, x, su, nx, cn, Wp, Wh, Wsu, Wm, Wx, b1, W2, b2, W3, b3)


def _pad_rows(a, n):
    return jnp.pad(a, ((0, n - a.shape[0]),) + ((0, 0),) * (a.ndim - 1))


def _pad_idx(idx, fill):
    p = jnp.full((E_PAD - E_EDGE,), fill, jnp.int32)
    return jnp.concatenate([idx.astype(jnp.int32), p]).reshape(E_PAD // IDX_LANES,
                                                               IDX_LANES)


def kernel(pos_state, pos_action, h, x, u, a2s_edge_index, a2s_dis,
           s2s_edge_index, s2s_dis,
           u2h_W1, u2h_b1, u2h_W2, u2h_b2, u2h_W3, u2h_b3,
           x2h_W1, x2h_b1, x2h_W2, x2h_b2, x2h_W3, x2h_b3,
           upd_W1, upd_b1, upd_W2, upd_b2, upd_W3, upd_b3):
    ps_p = _pad_rows(pos_state, NP)
    featA = _pad_rows(jnp.concatenate([pos_action, u], axis=1), NP)
    featS = _pad_rows(jnp.concatenate([pos_state, x, h], axis=1), NP)

    WA = jnp.concatenate([u2h_W1[0:2], u2h_W1[5:9]], axis=0)
    Wud = u2h_W1[2:4]
    w1d_u = u2h_W1[4:5]
    WS = jnp.concatenate([x2h_W1[0:2], x2h_W1[5:29]], axis=0)
    Wxd = x2h_W1[2:4]
    w1d_x = x2h_W1[4:5]

    TA, TS, TDU, TDX = _tc_project(
        featA, featS, ps_p, WA, u2h_b1[None, :], WS, x2h_b1[None, :], Wud, Wxd)

    sidxA = _pad_idx(a2s_edge_index[0], 0)
    didxA_g = _pad_idx(a2s_edge_index[1], 0)
    didxA_s = _pad_idx(a2s_edge_index[1], N_NODE)
    sidxS = _pad_idx(s2s_edge_index[0], 0)
    didxS_g = _pad_idx(s2s_edge_index[1], 0)
    didxS_s = _pad_idx(s2s_edge_index[1], N_NODE)

    disA = jnp.pad(a2s_dis, ((0, E_PAD - E_EDGE), (0, 0)))
    disS = jnp.pad(s2s_dis, ((0, E_PAD - E_EDGE), (0, 0)))

    zeros16 = jnp.zeros((ACC_PER_TILE, 16), F32)
    zeros1 = jnp.zeros((ACC_PER_TILE,), F32)

    gsA, gdA = _sc_gather(TA, TDU, sidxA, didxA_g)
    msgA = _tc_edge_mlp(gsA, gdA, disA, w1d_u,
                        u2h_W2, u2h_b2[None, :], u2h_W3, u2h_b3[None, :])
    sumsA, _cntA = _sc_scatter(msgA, didxA_s, zeros16, zeros1, with_counts=False)

    gsS, gdS = _sc_gather(TS, TDX, sidxS, didxS_g)
    msgS = _tc_edge_mlp(gsS, gdS, disS, w1d_x,
                        x2h_W2, x2h_b2[None, :], x2h_W3, x2h_b3[None, :])
    sumsS, cntS = _sc_scatter(msgS, didxS_s, zeros16, zeros1, with_counts=True)

    su = jnp.pad(sumsA[:, :N_NODE], ((0, 0), (0, NP - N_NODE), (0, 0)))
    nx = jnp.pad(sumsS[:, :N_NODE], ((0, 0), (0, NP - N_NODE), (0, 0)))
    cn = jnp.pad(cntS[:, :N_NODE], ((0, 0), (0, NP - N_NODE)))

    Wp = upd_W1[0:2]
    Wh = upd_W1[2:18]
    Wsu = upd_W1[18:34]
    Wm = upd_W1[34:50]
    Wx = upd_W1[50:58]

    out = _tc_final(ps_p, _pad_rows(h, NP), _pad_rows(x, NP), su, nx, cn,
                    Wp, Wh, Wsu, Wm, Wx, upd_b1[None, :],
                    upd_W2, upd_b2[None, :], upd_W3, upd_b3[None, :])
    return out[:N_NODE]
